# baseline (speedup 1.0000x reference)
import functools

import jax
import jax.numpy as jnp
from jax import lax
from jax.experimental import pallas as pl
from jax.experimental.pallas import tpu as pltpu

_B, _S, _D, _H = 4, 2048, 1024, 1024
_MT = 512
_TPR = _S // _MT
_NT = _B * _TPR


def _body(emb_ref, w1_ref, b1_ref, w2_ref, b2_ref,
          filt_ref, mask_ref, ek_ref,
          embscr, lgscr, thr_scr, pi_scr):
    s = pl.program_id(0)
    b = s // (2 * _TPR)
    ph = (s // _TPR) % 2
    t = s % _TPR
    tile = b * _TPR + t

    lane = lax.broadcasted_iota(jnp.int32, (1, _NT), 1)
    onehot = lane == tile
    rowlane = (lane // _TPR) == b

    @pl.when(ph == 0)
    def _score():
        x = jnp.dot(emb_ref[...], w1_ref[...], preferred_element_type=jnp.float32)
        x = jnp.maximum(x + b1_ref[...], 0.0)
        lg = jnp.dot(x, w2_ref[...], preferred_element_type=jnp.float32)
        lg = lg + b2_ref[...]
        embscr[pl.ds(tile * _MT, _MT), :] = emb_ref[...]
        lgscr[...] = jnp.where(onehot, lg, lgscr[...])

    @pl.when((ph == 0) & (t == _TPR - 1))
    def _row_epilogue():
        lgs = lgscr[...]
        rl = rowlane
        ek = jnp.sum(jnp.where(rl, jax.nn.sigmoid(lgs), 0.0),
                     axis=(0, 1), keepdims=True)[:, 0:1]
        ek_ref[...] = jnp.where(rl & (lane % _TPR == 0), ek, ek_ref[...])
        k = jnp.maximum(ek.astype(jnp.int32), 32)

        bits = lax.bitcast_convert_type(lgs, jnp.int32)
        key = jnp.where(bits < 0, bits ^ jnp.int32(0x7FFFFFFF), bits)

        def rowcount(pred):
            return jnp.sum(jnp.where(rl & pred, 1, 0),
                           axis=(0, 1), keepdims=True)[:, 0:1]

        nonneg = key >= 0
        cnt_nn = rowcount(nonneg)
        in_pos = k <= cnt_nn
        kk = jnp.where(in_pos, k, k - cnt_nn)
        cls = nonneg == in_pos
        m = key & jnp.int32(0x7FFFFFFF)
        p = jnp.zeros_like(k)
        for b_idx in range(30, -1, -1):
            q = p + jnp.int32(1 << b_idx)
            c = rowcount(cls & (m >= q))
            p = jnp.where(c >= kk, q, p)
        thr = jnp.where(in_pos, p, p | jnp.int32(-2147483648))

        c_gt = rowcount(key > thr)
        r = k - c_gt
        tie = key == thr
        sidx = (lax.broadcasted_iota(jnp.int32, (_MT, _NT), 0)
                + (lane % _TPR) * _MT)
        pi = jnp.zeros_like(k)
        for b_idx in range(11, -1, -1):
            qi = pi + jnp.int32(1 << b_idx)
            ci = rowcount(tie & (sidx < qi))
            pi = jnp.where(ci < r, qi, pi)

        thr_scr[...] = jnp.where(rl, jnp.broadcast_to(thr, (1, _NT)),
                                 thr_scr[...])
        pi_scr[...] = jnp.where(rl, jnp.broadcast_to(pi, (1, _NT)),
                                pi_scr[...])

    @pl.when(ph == 1)
    def _emit():
        ohf = onehot.astype(jnp.float32)
        lg = jnp.sum(lgscr[...] * ohf, axis=1, keepdims=True)
        thr = jnp.sum(thr_scr[...] * onehot.astype(jnp.int32),
                      axis=1, keepdims=True)
        pi = jnp.sum(pi_scr[...] * onehot.astype(jnp.int32),
                     axis=1, keepdims=True)
        bits = lax.bitcast_convert_type(lg, jnp.int32)
        key = jnp.where(bits < 0, bits ^ jnp.int32(0x7FFFFFFF), bits)
        sidx = lax.broadcasted_iota(jnp.int32, (_MT, 1), 0) + t * _MT
        hard = (key > thr) | ((key == thr) & (sidx <= pi))
        mk = hard.astype(jnp.float32)
        mask_ref[...] = mk
        filt_ref[...] = embscr[pl.ds(tile * _MT, _MT), :] * mk


def kernel(token_embeddings, W1, b1, W2, b2):
    emb2d = token_embeddings.reshape(_B * _S, _D)

    def in_idx(s):
        b = s // (2 * _TPR)
        ph = (s // _TPR) % 2
        t = s % _TPR
        return (jnp.where(ph == 0, b * _TPR + t, b * _TPR + _TPR - 1), 0)

    def out_idx(s):
        b = s // (2 * _TPR)
        ph = (s // _TPR) % 2
        t = s % _TPR
        return (jnp.where(ph == 0, jnp.maximum(b * _TPR - 1, 0), b * _TPR + t),
                0)

    filt, mask, ekv = pl.pallas_call(
        _body,
        grid=(2 * _NT,),
        in_specs=[
            pl.BlockSpec((_MT, _D), in_idx),
            pl.BlockSpec((_D, _H), lambda s: (0, 0)),
            pl.BlockSpec((1, _H), lambda s: (0, 0)),
            pl.BlockSpec((_H, 1), lambda s: (0, 0)),
            pl.BlockSpec((1, 1), lambda s: (0, 0)),
        ],
        out_specs=(
            pl.BlockSpec((_MT, _D), out_idx),
            pl.BlockSpec((_MT, 1), out_idx),
            pl.BlockSpec((1, _NT), lambda s: (0, 0)),
        ),
        out_shape=(
            jax.ShapeDtypeStruct((_B * _S, _D), jnp.float32),
            jax.ShapeDtypeStruct((_B * _S, 1), jnp.float32),
            jax.ShapeDtypeStruct((1, _NT), jnp.float32),
        ),
        scratch_shapes=[
            pltpu.VMEM((_B * _S, _D), jnp.float32),
            pltpu.VMEM((_MT, _NT), jnp.float32),
            pltpu.VMEM((1, _NT), jnp.int32),
            pltpu.VMEM((1, _NT), jnp.int32),
        ],
        compiler_params=pltpu.CompilerParams(
            dimension_semantics=("arbitrary",),
        ),
    )(emb2d, W1, b1.reshape(1, _H), W2, b2.reshape(1, 1))

    ek = ekv[0, :: _TPR]
    return filt.reshape(_B, _S, _D), mask.reshape(_B, _S), ek


# speedup vs baseline: 1.3211x; 1.3211x over previous
import jax
import jax.numpy as jnp
from jax import lax
from jax.experimental import pallas as pl
from jax.experimental.pallas import tpu as pltpu

_B, _S, _D, _H = 4, 2048, 1024, 1024
_MT = 512
_TPR = _S // _MT
_NT = _B * _TPR


def _score_body(emb_ref, w1_ref, b1_ref, w2_ref, b2_ref, lg_ref):
    x = jnp.dot(emb_ref[...], w1_ref[...], preferred_element_type=jnp.float32)
    x = jnp.maximum(x + b1_ref[...], 0.0)
    lg = jnp.dot(x, w2_ref[...], preferred_element_type=jnp.float32)
    lg_ref[...] = lg + b2_ref[...]


def _emit_body(lgq_ref, lgt_ref, emb_ref,
               filt_ref, mask_ref, ek_ref,
               thr_scr, pi_scr):
    s = pl.program_id(0)

    @pl.when(s == 0)
    def _scalars():
        lgs = lgq_ref[...]
        ek = jnp.sum(jax.nn.sigmoid(lgs), axis=1, keepdims=True)
        ek_ref[...] = ek
        k = jnp.maximum(ek.astype(jnp.int32), 32)

        bits = lax.bitcast_convert_type(lgs, jnp.int32)
        key = jnp.where(bits < 0, bits ^ jnp.int32(0x7FFFFFFF), bits)

        def rowcount(pred):
            return jnp.sum(pred.astype(jnp.int32), axis=1, keepdims=True)

        nonneg = key >= 0
        cnt_nn = rowcount(nonneg)
        in_pos = k <= cnt_nn
        kk = jnp.where(in_pos, k, k - cnt_nn)
        cls = nonneg == in_pos
        m = key & jnp.int32(0x7FFFFFFF)
        p = jnp.zeros_like(k)
        for b_idx in range(30, -1, -1):
            q = p + jnp.int32(1 << b_idx)
            c = rowcount(cls & (m >= q))
            p = jnp.where(c >= kk, q, p)
        thr = jnp.where(in_pos, p, p | jnp.int32(-2147483648))

        c_gt = rowcount(key > thr)
        r = k - c_gt
        tie = key == thr
        sidx = lax.broadcasted_iota(jnp.int32, (_B, _S), 1)
        pi = jnp.zeros_like(k)
        for b_idx in range(11, -1, -1):
            qi = pi + jnp.int32(1 << b_idx)
            ci = rowcount(tie & (sidx < qi))
            pi = jnp.where(ci < r, qi, pi)

        thr_scr[...] = thr
        pi_scr[...] = pi

    row = s // _TPR
    t = s % _TPR
    thr = thr_scr[pl.ds(row, 1), :]
    pi = pi_scr[pl.ds(row, 1), :]
    lg = lgt_ref[...]
    bits = lax.bitcast_convert_type(lg, jnp.int32)
    key = jnp.where(bits < 0, bits ^ jnp.int32(0x7FFFFFFF), bits)
    sidx = lax.broadcasted_iota(jnp.int32, (_MT, 1), 0) + t * _MT
    hard = (key > thr) | ((key == thr) & (sidx <= pi))
    mk = hard.astype(jnp.float32)
    mask_ref[...] = mk
    filt_ref[...] = emb_ref[...] * mk


def kernel(token_embeddings, W1, b1, W2, b2):
    emb2d = token_embeddings.reshape(_B * _S, _D)

    logits = pl.pallas_call(
        _score_body,
        grid=(_NT,),
        in_specs=[
            pl.BlockSpec((_MT, _D), lambda s: (s, 0)),
            pl.BlockSpec((_D, _H), lambda s: (0, 0)),
            pl.BlockSpec((1, _H), lambda s: (0, 0)),
            pl.BlockSpec((_H, 1), lambda s: (0, 0)),
            pl.BlockSpec((1, 1), lambda s: (0, 0)),
        ],
        out_specs=pl.BlockSpec((_MT, 1), lambda s: (s, 0)),
        out_shape=jax.ShapeDtypeStruct((_B * _S, 1), jnp.float32),
        compiler_params=pltpu.CompilerParams(
            dimension_semantics=("arbitrary",),
        ),
    )(emb2d, W1, b1.reshape(1, _H), W2, b2.reshape(1, 1))

    lgq = logits.reshape(_B, _S)

    filt, mask, ek = pl.pallas_call(
        _emit_body,
        grid=(_NT,),
        in_specs=[
            pl.BlockSpec((_B, _S), lambda s: (0, 0)),
            pl.BlockSpec((_MT, 1), lambda s: (s, 0)),
            pl.BlockSpec((_MT, _D), lambda s: (s, 0)),
        ],
        out_specs=(
            pl.BlockSpec((_MT, _D), lambda s: (s, 0)),
            pl.BlockSpec((_MT, 1), lambda s: (s, 0)),
            pl.BlockSpec((_B, 1), lambda s: (0, 0)),
        ),
        out_shape=(
            jax.ShapeDtypeStruct((_B * _S, _D), jnp.float32),
            jax.ShapeDtypeStruct((_B * _S, 1), jnp.float32),
            jax.ShapeDtypeStruct((_B, 1), jnp.float32),
        ),
        scratch_shapes=[
            pltpu.VMEM((_B, 1), jnp.int32),
            pltpu.VMEM((_B, 1), jnp.int32),
        ],
        compiler_params=pltpu.CompilerParams(
            dimension_semantics=("arbitrary",),
        ),
    )(lgq, logits, emb2d)

    return filt.reshape(_B, _S, _D), mask.reshape(_B, _S), ek[:, 0]
